# baseline (speedup 1.0000x reference)
import functools

import numpy as np

import jax
import jax.numpy as jnp
from jax.experimental import pallas as pl
from jax.experimental.pallas import tpu as pltpu


def _bilin_matrix(out_size, in_size):
    scale = in_size / out_size
    m = np.zeros((out_size, in_size), np.float32)
    for o in range(out_size):
        src = max((o + 0.5) * scale - 0.5, 0.0)
        i0 = min(int(np.floor(src)), in_size - 1)
        i1 = min(i0 + 1, in_size - 1)
        l1 = src - i0
        m[o, i0] += 1.0 - l1
        m[o, i1] += l1
    return m


def _fused_kernel(x_ref, skip_ref, m_ref, w1s_ref, b1_ref, w3c_ref, b3_ref,
                  mask_ref, o_ref, ypad_ref, y9_ref, *, ws, margin):
    c = o_ref.shape[0]
    s = o_ref.shape[1] * o_ref.shape[2]
    hxwx = m_ref.shape[0]
    cskip = skip_ref.shape[0]

    xb = x_ref[...].reshape(c, hxwx).astype(jnp.bfloat16)
    up = jnp.dot(xb, m_ref[...], preferred_element_type=jnp.float32)

    sb = jnp.dot(w1s_ref[...], skip_ref[...].reshape(cskip, s),
                 preferred_element_type=jnp.float32)
    sb = jnp.maximum(sb + b1_ref[...], 0.0)

    ypad_ref[:, :margin] = jnp.zeros((c, margin), ypad_ref.dtype)
    ypad_ref[:, s + margin:] = jnp.zeros((c, margin), ypad_ref.dtype)
    ypad_ref[:, margin:s + margin] = (up + sb).astype(ypad_ref.dtype)

    for k in range(9):
        dy = k // 3 - 1
        dx = k % 3 - 1
        d = dy * ws + dx
        ys = ypad_ref[:, pl.ds(margin + d, s)]
        if dx == -1:
            ys = ys * mask_ref[0]
        elif dx == 1:
            ys = ys * mask_ref[1]
        y9_ref[k * c:(k + 1) * c, :] = ys

    acc = jnp.dot(w3c_ref[...], y9_ref[...],
                  preferred_element_type=jnp.float32)
    out = jnp.maximum(acc + b3_ref[...], 0.0)
    o_ref[...] = out.reshape(o_ref.shape)


def kernel(skip, x, w1, w3, s1, b1, s3, b3):
    N, Cskip, Hs, Ws = skip.shape
    _, C, Hx, Wx = x.shape
    S = Hs * Ws
    margin = 128

    wh = _bilin_matrix(Hs, Hx)
    ww = _bilin_matrix(Ws, Wx)
    mfull = np.einsum("oi,pj->ijop", wh, ww).reshape(Hx * Wx, S)
    mfull = jnp.asarray(mfull, dtype=jnp.bfloat16)

    w1s = w1 * s1[:, None]
    w3c = (jnp.transpose(w3 * s3[:, None, None, None], (0, 2, 3, 1))
           .reshape(C, 9 * C).astype(jnp.bfloat16))
    b1c = b1[:, None]
    b3c = b3[:, None]

    ws_idx = np.arange(S) % Ws
    mask = np.stack([(ws_idx != 0), (ws_idx != Ws - 1)]).astype(np.float32)
    mask = jnp.asarray(mask.reshape(2, 1, S), dtype=jnp.bfloat16)

    out = pl.pallas_call(
        functools.partial(_fused_kernel, ws=Ws, margin=margin),
        out_shape=jax.ShapeDtypeStruct((N, C, Hs, Ws), jnp.float32),
        grid_spec=pltpu.PrefetchScalarGridSpec(
            num_scalar_prefetch=0,
            grid=(N,),
            in_specs=[
                pl.BlockSpec((None, C, Hx, Wx), lambda n: (n, 0, 0, 0)),
                pl.BlockSpec((None, Cskip, Hs, Ws), lambda n: (n, 0, 0, 0)),
                pl.BlockSpec((Hx * Wx, S), lambda n: (0, 0)),
                pl.BlockSpec((C, Cskip), lambda n: (0, 0)),
                pl.BlockSpec((C, 1), lambda n: (0, 0)),
                pl.BlockSpec((C, 9 * C), lambda n: (0, 0)),
                pl.BlockSpec((C, 1), lambda n: (0, 0)),
                pl.BlockSpec((2, 1, S), lambda n: (0, 0, 0)),
            ],
            out_specs=pl.BlockSpec((None, C, Hs, Ws), lambda n: (n, 0, 0, 0)),
            scratch_shapes=[pltpu.VMEM((C, S + 2 * margin), jnp.bfloat16),
                            pltpu.VMEM((9 * C, S), jnp.bfloat16)],
        ),
        compiler_params=pltpu.CompilerParams(
            dimension_semantics=("parallel",),
            allow_input_fusion=[True] * 8,
        ),
    )(x, skip, mfull, w1s, b1c, w3c, b3c, mask)

    return out


# speedup vs baseline: 1.0832x; 1.0832x over previous
import functools

import numpy as np

import jax
import jax.numpy as jnp
from jax.experimental import pallas as pl
from jax.experimental.pallas import tpu as pltpu


def _bilin_matrix(out_size, in_size):
    scale = in_size / out_size
    m = np.zeros((out_size, in_size), np.float32)
    for o in range(out_size):
        src = max((o + 0.5) * scale - 0.5, 0.0)
        i0 = min(int(np.floor(src)), in_size - 1)
        i1 = min(i0 + 1, in_size - 1)
        l1 = src - i0
        m[o, i0] += 1.0 - l1
        m[o, i1] += l1
    return m


def _fused_kernel(x_ref, skip_ref, m_ref, w1s_ref, b1_ref, w3c_ref, b3_ref,
                  mask_ref, o_ref, ypad_ref, y9_ref, *, ws, margin):
    c, s = o_ref.shape
    hxwx = m_ref.shape[0]

    xb = x_ref[...].reshape(c, hxwx)
    up = jnp.dot(xb, m_ref[...], preferred_element_type=jnp.float32)

    sb = jnp.dot(w1s_ref[...], skip_ref[...],
                 preferred_element_type=jnp.float32)
    sb = jnp.maximum(sb + b1_ref[...], 0.0)

    ypad_ref[:, :margin] = jnp.zeros((c, margin), ypad_ref.dtype)
    ypad_ref[:, s + margin:] = jnp.zeros((c, margin), ypad_ref.dtype)
    ypad_ref[:, margin:s + margin] = (up + sb).astype(ypad_ref.dtype)

    for k in range(9):
        dy = k // 3 - 1
        dx = k % 3 - 1
        d = dy * ws + dx
        ys = ypad_ref[:, pl.ds(margin + d, s)]
        if dx == -1:
            ys = ys * mask_ref[0]
        elif dx == 1:
            ys = ys * mask_ref[1]
        y9_ref[k * c:(k + 1) * c, :] = ys

    acc = jnp.dot(w3c_ref[...], y9_ref[...],
                  preferred_element_type=jnp.float32)
    o_ref[...] = jnp.maximum(acc + b3_ref[...], 0.0)


def kernel(skip, x, w1, w3, s1, b1, s3, b3):
    N, Cskip, Hs, Ws = skip.shape
    _, C, Hx, Wx = x.shape
    S = Hs * Ws
    margin = 128

    wh = _bilin_matrix(Hs, Hx)
    ww = _bilin_matrix(Ws, Wx)
    mfull = np.einsum("oi,pj->ijop", wh, ww).reshape(Hx * Wx, S)
    mfull = jnp.asarray(mfull, dtype=jnp.bfloat16)

    w1s = (w1 * s1[:, None]).astype(jnp.bfloat16)
    w3c = (jnp.transpose(w3 * s3[:, None, None, None], (0, 2, 3, 1))
           .reshape(C, 9 * C).astype(jnp.bfloat16))
    b1c = b1[:, None]
    b3c = b3[:, None]

    ws_idx = np.arange(S) % Ws
    mask = np.stack([(ws_idx != 0), (ws_idx != Ws - 1)]).astype(np.float32)
    mask = jnp.asarray(mask.reshape(2, 1, S), dtype=jnp.bfloat16)

    skip2 = skip.reshape(N, Cskip, S).astype(jnp.bfloat16)
    xb = x.astype(jnp.bfloat16)

    out = pl.pallas_call(
        functools.partial(_fused_kernel, ws=Ws, margin=margin),
        out_shape=jax.ShapeDtypeStruct((N, C, S), jnp.float32),
        grid_spec=pltpu.PrefetchScalarGridSpec(
            num_scalar_prefetch=0,
            grid=(N,),
            in_specs=[
                pl.BlockSpec((None, C, Hx, Wx), lambda n: (n, 0, 0, 0)),
                pl.BlockSpec((None, Cskip, S), lambda n: (n, 0, 0)),
                pl.BlockSpec((Hx * Wx, S), lambda n: (0, 0)),
                pl.BlockSpec((C, Cskip), lambda n: (0, 0)),
                pl.BlockSpec((C, 1), lambda n: (0, 0)),
                pl.BlockSpec((C, 9 * C), lambda n: (0, 0)),
                pl.BlockSpec((C, 1), lambda n: (0, 0)),
                pl.BlockSpec((2, 1, S), lambda n: (0, 0, 0)),
            ],
            out_specs=pl.BlockSpec((None, C, S), lambda n: (n, 0, 0)),
            scratch_shapes=[pltpu.VMEM((C, S + 2 * margin), jnp.bfloat16),
                            pltpu.VMEM((9 * C, S), jnp.bfloat16)],
        ),
        compiler_params=pltpu.CompilerParams(dimension_semantics=("parallel",)),
    )(xb, skip2, mfull, w1s, b1c, w3c, b3c, mask)

    return out.reshape(N, C, Hs, Ws)
